# baseline (speedup 1.0000x reference)
import jax
import jax.numpy as jnp
from jax import lax
from jax.experimental import pallas as pl

_B, _N, _DEG = 8, 256, 32
_C, _DE, _R = 256, 16, 32
_E = _B * _N * _DEG
_f32 = jnp.float32


def _mm_body(x_ref, w_ref, o_ref):
    o_ref[...] = jnp.dot(x_ref[...], w_ref[...], preferred_element_type=_f32)


def _mm(x, w, bm=None):
    M, K = x.shape
    Nw = w.shape[1]
    if bm is None:
        bm = M
    return pl.pallas_call(
        _mm_body,
        grid=(M // bm,),
        in_specs=[pl.BlockSpec((bm, K), lambda i: (i, 0)),
                  pl.BlockSpec((K, Nw), lambda i: (0, 0))],
        out_specs=pl.BlockSpec((bm, Nw), lambda i: (i, 0)),
        out_shape=jax.ShapeDtypeStruct((M, Nw), _f32),
    )(x, w)


def _p1_body(xws_ref, aggs_ref, xwt_ref, aggt_ref, shat_ref):
    hs = jnp.maximum(xws_ref[0] + aggs_ref[0], 0.0)
    ht = jnp.maximum(xwt_ref[0] + aggt_ref[0], 0.0)
    shat_ref[0] = lax.dot_general(hs, ht, (((1,), (1,)), ((), ())),
                                  preferred_element_type=_f32)


def _bspec(shape):
    nd = len(shape)
    return pl.BlockSpec((1,) + shape[1:], lambda b: (b,) + (0,) * (nd - 1))


def _cspec(shape):
    nd = len(shape)
    return pl.BlockSpec(shape, lambda b: (0,) * nd)


def _p1(xws, aggs, xwt, aggt):
    args = (xws, aggs, xwt, aggt)
    return pl.pallas_call(
        _p1_body,
        grid=(_B,),
        in_specs=[_bspec(a.shape) for a in args],
        out_specs=_bspec((_B, _N, _N)),
        out_shape=jax.ShapeDtypeStruct((_B, _N, _N), _f32),
    )(*args)


def _p2_body(s_ref, rs_ref, wm_ref, w_ref, y_ref, rw_ref):
    r_t = lax.dot_general(s_ref[0], rs_ref[0], (((0,), (0,)), ((), ())),
                          preferred_element_type=_f32)
    y_ref[0] = jnp.dot(r_t, wm_ref[...], preferred_element_type=_f32)
    rw_ref[0] = jnp.dot(r_t, w_ref[...], preferred_element_type=_f32)


def _p2(S, r_s3, Wm, W):
    return pl.pallas_call(
        _p2_body,
        grid=(_B,),
        in_specs=[_bspec(S.shape), _bspec(r_s3.shape),
                  _cspec(Wm.shape), _cspec(W.shape)],
        out_specs=[_bspec((_B, _N, _R)), _bspec((_B, _N, _R))],
        out_shape=[jax.ShapeDtypeStruct((_B, _N, _R), _f32),
                   jax.ShapeDtypeStruct((_B, _N, _R), _f32)],
    )(S, r_s3, Wm, W)


_JC = 64


def _p3_body(rws_ref, aggs_ref, rwt_ref, aggt_ref,
             w1_ref, b1_ref, w2_ref, b2_ref, shat_ref, out_ref):
    o_s = jnp.maximum(rws_ref[0] + aggs_ref[0], 0.0)
    o_t = jnp.maximum(rwt_ref[0] + aggt_ref[0], 0.0)
    w1 = w1_ref[...]
    b1 = b1_ref[...]
    w2 = w2_ref[...]
    b2 = b2_ref[0, 0]
    cols = []
    for jc in range(_N // _JC):
        ot_c = o_t[jc * _JC:(jc + 1) * _JC, :]
        D = o_s[:, None, :] - ot_c[None, :, :]
        D2 = D.reshape(_N * _JC, _R)
        h1 = jnp.maximum(jnp.dot(D2, w1, preferred_element_type=_f32)
                         + b1, 0.0)
        mc = jnp.dot(h1, w2, preferred_element_type=_f32) + b2
        cols.append(mc.reshape(_N, _JC))
    m = jnp.concatenate(cols, axis=1)
    out_ref[0] = shat_ref[0] + m


def _p3(rws, aggs, rwt, aggt, w1, b1, w2, b2, shat):
    barr = (rws, aggs, rwt, aggt)
    carr = (w1, b1, w2, b2)
    return pl.pallas_call(
        _p3_body,
        grid=(_B,),
        in_specs=[_bspec(a.shape) for a in barr]
        + [_cspec(a.shape) for a in carr]
        + [_bspec(shat.shape)],
        out_specs=_bspec((_B, _N, _N)),
        out_shape=jax.ShapeDtypeStruct((_B, _N, _N), _f32),
    )(*barr, *carr, shat)


def _masked_softmax(src, mask):
    out = jnp.where(mask, src, -jnp.inf)
    out = jax.nn.softmax(out, axis=-1)
    return jnp.where(mask, out, 0.0)


def _agg(y, eaWe, ei):
    msg = y[ei[0]] + eaWe
    return jax.ops.segment_sum(msg, ei[1], num_segments=_B * _N)


def kernel(x_s, edge_index_s, edge_attr_s, batch_s, x_t, edge_index_t,
           edge_attr_t, batch_t, psi1_W, psi1_Wm, psi1_We,
           psiA_W, psiA_Wm, psiA_We, psiB_W, psiB_Wm, psiB_We,
           mlp_W1, mlp_b1, mlp_W2, mlp_b2, sum_weights):
    y1s = _mm(x_s, psi1_Wm)
    y1t = _mm(x_t, psi1_Wm)
    xW1s = _mm(x_s, psi1_W)
    xW1t = _mm(x_t, psi1_W)
    eaWe1_s = _mm(edge_attr_s, psi1_We, bm=4096)
    eaWe1_t = _mm(edge_attr_t, psi1_We, bm=4096)
    eaWeA_s = _mm(edge_attr_s, psiA_We, bm=8192)
    eaWeA_t = _mm(edge_attr_t, psiA_We, bm=8192)
    eaWeB_s = _mm(edge_attr_s, psiB_We, bm=8192)
    eaWeB_t = _mm(edge_attr_t, psiB_We, bm=8192)

    rkey = jax.random.key(42)
    rA = jax.random.normal(jax.random.fold_in(rkey, 0), (_B, _N, _R), _f32)
    rB = jax.random.normal(jax.random.fold_in(rkey, 1), (_B, _N, _R), _f32)
    rA2 = rA.reshape(_B * _N, _R)
    rB2 = rB.reshape(_B * _N, _R)
    yA_s = _mm(rA2, psiA_Wm)
    rAWA = _mm(rA2, psiA_W)
    yB_s = _mm(rB2, psiB_Wm)
    rBWB = _mm(rB2, psiB_W)

    agg1_s = _agg(y1s, eaWe1_s, edge_index_s)
    agg1_t = _agg(y1t, eaWe1_t, edge_index_t)

    S_hat = _p1(xW1s.reshape(_B, _N, _C), agg1_s.reshape(_B, _N, _C),
                xW1t.reshape(_B, _N, _C), agg1_t.reshape(_B, _N, _C))

    s_mask = jnp.ones((_B, _N), dtype=bool)
    S_mask = s_mask[:, :, None] & s_mask[:, None, :]
    S_0 = _masked_softmax(S_hat, S_mask).reshape(_B * _N, _N)

    b1r = mlp_b1.reshape(1, _R)
    b2r = mlp_b2.reshape(1, 1)

    def stage(S_hat, S, r3, yS, rWS, We_ea_s, We_ea_t, Wm, W):
        S3 = S.reshape(_B, _N, _N)
        y_t, rtW = _p2(S3, r3, Wm, W)
        agg_s = _agg(yS, We_ea_s, edge_index_s)
        agg_t = _agg(y_t.reshape(_B * _N, _R), We_ea_t, edge_index_t)
        return _p3(rWS.reshape(_B, _N, _R), agg_s.reshape(_B, _N, _R),
                   rtW, agg_t.reshape(_B, _N, _R),
                   mlp_W1, b1r, mlp_W2, b2r, S_hat)

    S_A = _masked_softmax(S_hat, S_mask)
    S_hat = stage(S_hat, S_A, rA, yA_s, rAWA, eaWeA_s, eaWeA_t,
                  psiA_Wm, psiA_W)
    S_1 = _masked_softmax(S_hat, S_mask).reshape(_B * _N, _N)
    S_hat = stage(S_hat, S_1.reshape(_B, _N, _N), rB, yB_s, rBWB,
                  eaWeB_s, eaWeB_t, psiB_Wm, psiB_W)
    S_2 = _masked_softmax(S_hat, S_mask).reshape(_B * _N, _N)

    S_final = sum_weights[0] * S_0
    S_final = S_final + sum_weights[1] * S_1
    S_final = S_final + sum_weights[2] * S_2
    S_final = jax.nn.softmax(S_final, axis=-1)
    return (S_0, S_final)


# speedup vs baseline: 1.0008x; 1.0008x over previous
import jax
import jax.numpy as jnp
from jax import lax
from jax.experimental import pallas as pl

_B, _N, _DEG = 8, 256, 32
_C, _DE, _R = 256, 16, 32
_E = _B * _N * _DEG
_f32 = jnp.float32


def _mm_body(x_ref, w_ref, o_ref):
    o_ref[...] = jnp.dot(x_ref[...], w_ref[...], preferred_element_type=_f32)


def _mm(x, w, bm=None):
    M, K = x.shape
    Nw = w.shape[1]
    if bm is None:
        bm = M
    return pl.pallas_call(
        _mm_body,
        grid=(M // bm,),
        in_specs=[pl.BlockSpec((bm, K), lambda i: (i, 0)),
                  pl.BlockSpec((K, Nw), lambda i: (0, 0))],
        out_specs=pl.BlockSpec((bm, Nw), lambda i: (i, 0)),
        out_shape=jax.ShapeDtypeStruct((M, Nw), _f32),
    )(x, w)


def _p1_body(xws_ref, aggs_ref, xwt_ref, aggt_ref, shat_ref):
    hs = jnp.maximum(xws_ref[0] + aggs_ref[0], 0.0)
    ht = jnp.maximum(xwt_ref[0] + aggt_ref[0], 0.0)
    shat_ref[0] = lax.dot_general(hs, ht, (((1,), (1,)), ((), ())),
                                  preferred_element_type=_f32)


def _bspec(shape):
    nd = len(shape)
    return pl.BlockSpec((1,) + shape[1:], lambda b: (b,) + (0,) * (nd - 1))


def _cspec(shape):
    nd = len(shape)
    return pl.BlockSpec(shape, lambda b: (0,) * nd)


def _p1(xws, aggs, xwt, aggt):
    args = (xws, aggs, xwt, aggt)
    return pl.pallas_call(
        _p1_body,
        grid=(_B,),
        in_specs=[_bspec(a.shape) for a in args],
        out_specs=_bspec((_B, _N, _N)),
        out_shape=jax.ShapeDtypeStruct((_B, _N, _N), _f32),
    )(*args)


def _p2_body(s_ref, rs_ref, wm_ref, w_ref, y_ref, rw_ref):
    r_t = lax.dot_general(s_ref[0], rs_ref[0], (((0,), (0,)), ((), ())),
                          preferred_element_type=_f32)
    y_ref[0] = jnp.dot(r_t, wm_ref[...], preferred_element_type=_f32)
    rw_ref[0] = jnp.dot(r_t, w_ref[...], preferred_element_type=_f32)


def _p2(S, r_s3, Wm, W):
    return pl.pallas_call(
        _p2_body,
        grid=(_B,),
        in_specs=[_bspec(S.shape), _bspec(r_s3.shape),
                  _cspec(Wm.shape), _cspec(W.shape)],
        out_specs=[_bspec((_B, _N, _R)), _bspec((_B, _N, _R))],
        out_shape=[jax.ShapeDtypeStruct((_B, _N, _R), _f32),
                   jax.ShapeDtypeStruct((_B, _N, _R), _f32)],
    )(S, r_s3, Wm, W)


_JC = 64


def _p3_body(rws_ref, aggs_ref, rwt_ref, aggt_ref,
             w1_ref, b1_ref, w2_ref, b2_ref, shat_ref, out_ref):
    o_s = jnp.maximum(rws_ref[0] + aggs_ref[0], 0.0)
    o_t = jnp.maximum(rwt_ref[0] + aggt_ref[0], 0.0)
    w1 = w1_ref[...]
    b1 = b1_ref[...]
    w2 = w2_ref[...]
    b2 = b2_ref[0, 0]
    cols = []
    for jc in range(_N // _JC):
        ot_c = o_t[jc * _JC:(jc + 1) * _JC, :]
        D = o_s[:, None, :] - ot_c[None, :, :]
        D2 = D.reshape(_N * _JC, _R)
        h1 = jnp.maximum(jnp.dot(D2, w1, preferred_element_type=_f32)
                         + b1, 0.0)
        mc = jnp.dot(h1, w2, preferred_element_type=_f32) + b2
        cols.append(mc.reshape(_N, _JC))
    m = jnp.concatenate(cols, axis=1)
    out_ref[0] = shat_ref[0] + m


def _p3(rws, aggs, rwt, aggt, w1, b1, w2, b2, shat):
    barr = (rws, aggs, rwt, aggt)
    carr = (w1, b1, w2, b2)
    return pl.pallas_call(
        _p3_body,
        grid=(_B,),
        in_specs=[_bspec(a.shape) for a in barr]
        + [_cspec(a.shape) for a in carr]
        + [_bspec(shat.shape)],
        out_specs=_bspec((_B, _N, _N)),
        out_shape=jax.ShapeDtypeStruct((_B, _N, _N), _f32),
    )(*barr, *carr, shat)


def _masked_softmax(src, mask):
    out = jnp.where(mask, src, -jnp.inf)
    out = jax.nn.softmax(out, axis=-1)
    return jnp.where(mask, out, 0.0)


def _agg(y, eaWe, ei):
    msg = y[ei[0]] + eaWe
    return jax.ops.segment_sum(msg, ei[1], num_segments=_B * _N)


def kernel(x_s, edge_index_s, edge_attr_s, batch_s, x_t, edge_index_t,
           edge_attr_t, batch_t, psi1_W, psi1_Wm, psi1_We,
           psiA_W, psiA_Wm, psiA_We, psiB_W, psiB_Wm, psiB_We,
           mlp_W1, mlp_b1, mlp_W2, mlp_b2, sum_weights):
    y1s = _mm(x_s, psi1_Wm)
    y1t = _mm(x_t, psi1_Wm)
    xW1s = _mm(x_s, psi1_W)
    xW1t = _mm(x_t, psi1_W)
    eaWe1_s = _mm(edge_attr_s, psi1_We, bm=4096)
    eaWe1_t = _mm(edge_attr_t, psi1_We, bm=4096)
    eaWeA_s = _mm(edge_attr_s, psiA_We, bm=8192)
    eaWeA_t = _mm(edge_attr_t, psiA_We, bm=8192)
    eaWeB_s = _mm(edge_attr_s, psiB_We, bm=8192)
    eaWeB_t = _mm(edge_attr_t, psiB_We, bm=8192)

    rkey = jax.random.key(42)
    rA = jax.random.normal(jax.random.fold_in(rkey, 0), (_B, _N, _R), _f32)
    rB = jax.random.normal(jax.random.fold_in(rkey, 1), (_B, _N, _R), _f32)
    rA2 = rA.reshape(_B * _N, _R)
    rB2 = rB.reshape(_B * _N, _R)
    yA_s = _mm(rA2, psiA_Wm)
    rAWA = _mm(rA2, psiA_W)
    yB_s = _mm(rB2, psiB_Wm)
    rBWB = _mm(rB2, psiB_W)

    aggA_s = _agg(yA_s, eaWeA_s, edge_index_s)
    aggB_s = _agg(yB_s, eaWeB_s, edge_index_s)

    agg1_t = _agg(y1t, eaWe1_t, edge_index_t)
    agg1_s = _agg(y1s, eaWe1_s, edge_index_s)

    S_hat = _p1(xW1s.reshape(_B, _N, _C), agg1_s.reshape(_B, _N, _C),
                xW1t.reshape(_B, _N, _C), agg1_t.reshape(_B, _N, _C))

    s_mask = jnp.ones((_B, _N), dtype=bool)
    S_mask = s_mask[:, :, None] & s_mask[:, None, :]
    S_0 = _masked_softmax(S_hat, S_mask).reshape(_B * _N, _N)

    b1r = mlp_b1.reshape(1, _R)
    b2r = mlp_b2.reshape(1, 1)

    def stage(S_hat, S, r3, agg_s, rWS, We_ea_t, Wm, W):
        S3 = S.reshape(_B, _N, _N)
        y_t, rtW = _p2(S3, r3, Wm, W)
        agg_t = _agg(y_t.reshape(_B * _N, _R), We_ea_t, edge_index_t)
        return _p3(rWS.reshape(_B, _N, _R), agg_s.reshape(_B, _N, _R),
                   rtW, agg_t.reshape(_B, _N, _R),
                   mlp_W1, b1r, mlp_W2, b2r, S_hat)

    S_A = _masked_softmax(S_hat, S_mask)
    S_hat = stage(S_hat, S_A, rA, aggA_s, rAWA, eaWeA_t,
                  psiA_Wm, psiA_W)
    S_1 = _masked_softmax(S_hat, S_mask).reshape(_B * _N, _N)
    S_hat = stage(S_hat, S_1.reshape(_B, _N, _N), rB, aggB_s, rBWB,
                  eaWeB_t, psiB_Wm, psiB_W)
    S_2 = _masked_softmax(S_hat, S_mask).reshape(_B * _N, _N)

    S_final = sum_weights[0] * S_0
    S_final = S_final + sum_weights[1] * S_1
    S_final = S_final + sum_weights[2] * S_2
    S_final = jax.nn.softmax(S_final, axis=-1)
    return (S_0, S_final)


# speedup vs baseline: 1.4899x; 1.4887x over previous
import jax
import jax.numpy as jnp
from jax import lax
from jax.experimental import pallas as pl

_B, _N, _DEG = 8, 256, 32
_C, _DE, _R = 256, 16, 32
_E = _B * _N * _DEG
_f32 = jnp.float32


def _mm_body(x_ref, w_ref, o_ref):
    o_ref[...] = jnp.dot(x_ref[...], w_ref[...], preferred_element_type=_f32)


def _mm(x, w, bm=None):
    M, K = x.shape
    Nw = w.shape[1]
    if bm is None:
        bm = M
    return pl.pallas_call(
        _mm_body,
        grid=(M // bm,),
        in_specs=[pl.BlockSpec((bm, K), lambda i: (i, 0)),
                  pl.BlockSpec((K, Nw), lambda i: (0, 0))],
        out_specs=pl.BlockSpec((bm, Nw), lambda i: (i, 0)),
        out_shape=jax.ShapeDtypeStruct((M, Nw), _f32),
    )(x, w)


def _p1_body(xws_ref, aggs_ref, xwt_ref, aggt_ref, shat_ref):
    hs = jnp.maximum(xws_ref[0] + aggs_ref[0], 0.0)
    ht = jnp.maximum(xwt_ref[0] + aggt_ref[0], 0.0)
    shat_ref[0] = lax.dot_general(hs, ht, (((1,), (1,)), ((), ())),
                                  preferred_element_type=_f32)


def _bspec(shape):
    nd = len(shape)
    return pl.BlockSpec((1,) + shape[1:], lambda b: (b,) + (0,) * (nd - 1))


def _cspec(shape):
    nd = len(shape)
    return pl.BlockSpec(shape, lambda b: (0,) * nd)


def _p1(xws, aggs, xwt, aggt):
    args = (xws, aggs, xwt, aggt)
    return pl.pallas_call(
        _p1_body,
        grid=(_B,),
        in_specs=[_bspec(a.shape) for a in args],
        out_specs=_bspec((_B, _N, _N)),
        out_shape=jax.ShapeDtypeStruct((_B, _N, _N), _f32),
    )(*args)


def _p2_body(s_ref, rs_ref, wm_ref, w_ref, y_ref, rw_ref):
    r_t = lax.dot_general(s_ref[0], rs_ref[0], (((0,), (0,)), ((), ())),
                          preferred_element_type=_f32)
    y_ref[0] = jnp.dot(r_t, wm_ref[...], preferred_element_type=_f32)
    rw_ref[0] = jnp.dot(r_t, w_ref[...], preferred_element_type=_f32)


def _p2(S, r_s3, Wm, W):
    return pl.pallas_call(
        _p2_body,
        grid=(_B,),
        in_specs=[_bspec(S.shape), _bspec(r_s3.shape),
                  _cspec(Wm.shape), _cspec(W.shape)],
        out_specs=[_bspec((_B, _N, _R)), _bspec((_B, _N, _R))],
        out_shape=[jax.ShapeDtypeStruct((_B, _N, _R), _f32),
                   jax.ShapeDtypeStruct((_B, _N, _R), _f32)],
    )(S, r_s3, Wm, W)


_JC = 64


def _p3_body(rws_ref, aggs_ref, rwt_ref, aggt_ref,
             w1_ref, b1_ref, w2_ref, b2_ref, shat_ref, out_ref):
    o_s = jnp.maximum(rws_ref[0] + aggs_ref[0], 0.0)
    o_t = jnp.maximum(rwt_ref[0] + aggt_ref[0], 0.0)
    w1 = w1_ref[...]
    b1 = b1_ref[...]
    w2 = w2_ref[...]
    b2 = b2_ref[0, 0]
    cols = []
    for jc in range(_N // _JC):
        ot_c = o_t[jc * _JC:(jc + 1) * _JC, :]
        D = o_s[:, None, :] - ot_c[None, :, :]
        D2 = D.reshape(_N * _JC, _R)
        h1 = jnp.maximum(jnp.dot(D2, w1, preferred_element_type=_f32)
                         + b1, 0.0)
        mc = jnp.dot(h1, w2, preferred_element_type=_f32) + b2
        cols.append(mc.reshape(_N, _JC))
    m = jnp.concatenate(cols, axis=1)
    out_ref[0] = shat_ref[0] + m


def _p3(rws, aggs, rwt, aggt, w1, b1, w2, b2, shat):
    barr = (rws, aggs, rwt, aggt)
    carr = (w1, b1, w2, b2)
    return pl.pallas_call(
        _p3_body,
        grid=(_B,),
        in_specs=[_bspec(a.shape) for a in barr]
        + [_cspec(a.shape) for a in carr]
        + [_bspec(shat.shape)],
        out_specs=_bspec((_B, _N, _N)),
        out_shape=jax.ShapeDtypeStruct((_B, _N, _N), _f32),
    )(*barr, *carr, shat)


_EC = 2048


def _pg_body(src_ref, y_ref, o_ref):
    s = src_ref[0, 0, 0]
    ohT = (lax.broadcasted_iota(jnp.int32, (_N, _EC), 0)
           == s[None, :]).astype(_f32)
    o_ref[0] = lax.dot_general(ohT, y_ref[0], (((0,), (0,)), ((), ())),
                               preferred_element_type=_f32,
                               precision=lax.Precision.HIGHEST)


def _gather(y2d, src_local):
    C = y2d.shape[1]
    y3 = y2d.reshape(_B, _N, C)
    nj = (_N * _DEG) // _EC
    src3 = src_local.reshape(_B, nj, 1, _EC)
    out = pl.pallas_call(
        _pg_body,
        grid=(_B, nj),
        in_specs=[pl.BlockSpec((1, 1, 1, _EC), lambda b, j: (b, j, 0, 0)),
                  pl.BlockSpec((1, _N, C), lambda b, j: (b, 0, 0))],
        out_specs=pl.BlockSpec((1, _EC, C), lambda b, j: (b * nj + j, 0, 0)),
        out_shape=jax.ShapeDtypeStruct((_B * nj, _EC, C), _f32),
    )(src3, y3)
    return out.reshape(_E, C)


def _masked_softmax(src, mask):
    out = jnp.where(mask, src, -jnp.inf)
    out = jax.nn.softmax(out, axis=-1)
    return jnp.where(mask, out, 0.0)


def _agg(y, eaWe, ei, src_local):
    msg = _gather(y, src_local) + eaWe
    return jax.ops.segment_sum(msg, ei[1], num_segments=_B * _N)


def kernel(x_s, edge_index_s, edge_attr_s, batch_s, x_t, edge_index_t,
           edge_attr_t, batch_t, psi1_W, psi1_Wm, psi1_We,
           psiA_W, psiA_Wm, psiA_We, psiB_W, psiB_Wm, psiB_We,
           mlp_W1, mlp_b1, mlp_W2, mlp_b2, sum_weights):
    y1s = _mm(x_s, psi1_Wm)
    y1t = _mm(x_t, psi1_Wm)
    xW1s = _mm(x_s, psi1_W)
    xW1t = _mm(x_t, psi1_W)
    eaWe1_s = _mm(edge_attr_s, psi1_We, bm=4096)
    eaWe1_t = _mm(edge_attr_t, psi1_We, bm=4096)
    eaWeA_s = _mm(edge_attr_s, psiA_We, bm=8192)
    eaWeA_t = _mm(edge_attr_t, psiA_We, bm=8192)
    eaWeB_s = _mm(edge_attr_s, psiB_We, bm=8192)
    eaWeB_t = _mm(edge_attr_t, psiB_We, bm=8192)

    rkey = jax.random.key(42)
    rA = jax.random.normal(jax.random.fold_in(rkey, 0), (_B, _N, _R), _f32)
    rB = jax.random.normal(jax.random.fold_in(rkey, 1), (_B, _N, _R), _f32)
    rA2 = rA.reshape(_B * _N, _R)
    rB2 = rB.reshape(_B * _N, _R)
    yA_s = _mm(rA2, psiA_Wm)
    rAWA = _mm(rA2, psiA_W)
    yB_s = _mm(rB2, psiB_Wm)
    rBWB = _mm(rB2, psiB_W)

    src_s = edge_index_s[0] % _N
    src_t = edge_index_t[0] % _N

    aggA_s = _agg(yA_s, eaWeA_s, edge_index_s, src_s)
    aggB_s = _agg(yB_s, eaWeB_s, edge_index_s, src_s)

    agg1_t = _agg(y1t, eaWe1_t, edge_index_t, src_t)
    agg1_s = _agg(y1s, eaWe1_s, edge_index_s, src_s)

    S_hat = _p1(xW1s.reshape(_B, _N, _C), agg1_s.reshape(_B, _N, _C),
                xW1t.reshape(_B, _N, _C), agg1_t.reshape(_B, _N, _C))

    s_mask = jnp.ones((_B, _N), dtype=bool)
    S_mask = s_mask[:, :, None] & s_mask[:, None, :]
    S_0 = _masked_softmax(S_hat, S_mask).reshape(_B * _N, _N)

    b1r = mlp_b1.reshape(1, _R)
    b2r = mlp_b2.reshape(1, 1)

    def stage(S_hat, S, r3, agg_s, rWS, We_ea_t, Wm, W):
        S3 = S.reshape(_B, _N, _N)
        y_t, rtW = _p2(S3, r3, Wm, W)
        agg_t = _agg(y_t.reshape(_B * _N, _R), We_ea_t, edge_index_t, src_t)
        return _p3(rWS.reshape(_B, _N, _R), agg_s.reshape(_B, _N, _R),
                   rtW, agg_t.reshape(_B, _N, _R),
                   mlp_W1, b1r, mlp_W2, b2r, S_hat)

    S_A = _masked_softmax(S_hat, S_mask)
    S_hat = stage(S_hat, S_A, rA, aggA_s, rAWA, eaWeA_t,
                  psiA_Wm, psiA_W)
    S_1 = _masked_softmax(S_hat, S_mask).reshape(_B * _N, _N)
    S_hat = stage(S_hat, S_1.reshape(_B, _N, _N), rB, aggB_s, rBWB,
                  eaWeB_t, psiB_Wm, psiB_W)
    S_2 = _masked_softmax(S_hat, S_mask).reshape(_B * _N, _N)

    S_final = sum_weights[0] * S_0
    S_final = S_final + sum_weights[1] * S_1
    S_final = S_final + sum_weights[2] * S_2
    S_final = jax.nn.softmax(S_final, axis=-1)
    return (S_0, S_final)
